# initial kernel scaffold (unmeasured)
import functools

import jax
import jax.numpy as jnp
from jax import lax
from jax.experimental import pallas as pl
from jax.experimental.pallas import tpu as pltpu

N_DEV = 8


def kernel(O, Wo):
    b, s, h, d = O.shape
    hd = h * d
    n_out = Wo.shape[1]
    s_chunk = s // N_DEV
    O2 = O.reshape(b, s, hd)

    def body(o_hbm, wo_ref, out_ref, obuf, comm, obuf_sems, send_sems,
             recv_sems, credit_sem):
        my = lax.axis_index("i")
        left = (my - 1) % N_DEV
        right = (my + 1) % N_DEV

        barrier_sem = pltpu.get_barrier_semaphore()
        for nbr in (left, right):
            pl.semaphore_signal(barrier_sem, inc=1, device_id=(nbr,),
                                device_id_type=pl.DeviceIdType.MESH)
        pl.semaphore_wait(barrier_sem, 2)

        def load_chunk(c, slot):
            cp = pltpu.make_async_copy(
                o_hbm.at[:, pl.ds(c * s_chunk, s_chunk), :],
                obuf.at[slot],
                obuf_sems.at[slot],
            )
            cp.start()
            return cp

        def partial(slot, bi):
            return jnp.dot(obuf[slot, bi], wo_ref[...],
                           preferred_element_type=jnp.float32)

        c0 = (my - 1) % N_DEV
        load_chunk(c0, 0).wait()
        for bi in range(b):
            comm[0, bi] = partial(0, bi)

        for t in range(N_DEV - 1):
            send_slot = t % 2
            recv_slot = (t + 1) % 2
            if t >= 1:
                pl.semaphore_wait(credit_sem, 1)
            rdma = pltpu.make_async_remote_copy(
                src_ref=comm.at[send_slot],
                dst_ref=comm.at[recv_slot],
                send_sem=send_sems.at[send_slot],
                recv_sem=recv_sems.at[recv_slot],
                device_id=(right,),
                device_id_type=pl.DeviceIdType.MESH,
            )
            rdma.start()
            c_recv = (my - 2 - t) % N_DEV
            cp = load_chunk(c_recv, recv_slot)
            cp.wait()
            rdma.wait()
            if t < N_DEV - 2:
                pl.semaphore_signal(credit_sem, inc=1, device_id=(left,),
                                    device_id_type=pl.DeviceIdType.MESH)
                for bi in range(b):
                    comm[recv_slot, bi] += partial(recv_slot, bi)
            else:
                for bi in range(b):
                    out_ref[bi] = comm[recv_slot, bi] + partial(recv_slot, bi)

        @functools.partial(pl.run_scoped,
                           exit_sem=pltpu.SemaphoreType.REGULAR)
        def _(exit_sem):
            for nbr in (left, right):
                pl.semaphore_signal(exit_sem, inc=1, device_id=(nbr,),
                                    device_id_type=pl.DeviceIdType.MESH)
            pl.semaphore_wait(exit_sem, 2)

    return pl.pallas_call(
        body,
        out_shape=jax.ShapeDtypeStruct((b, s_chunk, n_out), jnp.float32),
        in_specs=[
            pl.BlockSpec(memory_space=pltpu.ANY),
            pl.BlockSpec(memory_space=pltpu.VMEM),
        ],
        out_specs=pl.BlockSpec(memory_space=pltpu.VMEM),
        scratch_shapes=[
            pltpu.VMEM((2, b, s_chunk, hd), jnp.float32),
            pltpu.VMEM((2, b, s_chunk, n_out), jnp.float32),
            pltpu.SemaphoreType.DMA((2,)),
            pltpu.SemaphoreType.DMA((2,)),
            pltpu.SemaphoreType.DMA((2,)),
            pltpu.SemaphoreType.REGULAR,
        ],
        compiler_params=pltpu.CompilerParams(collective_id=0),
    )(O2, Wo)


# baseline (device time: 1362312 ns/iter reference)
import functools

import jax
import jax.numpy as jnp
from jax import lax
from jax.experimental import pallas as pl
from jax.experimental.pallas import tpu as pltpu

N_DEV = 8


def kernel(O, Wo):
    b, s, h, d = O.shape
    hd = h * d
    n_out = Wo.shape[1]
    s_chunk = s // N_DEV
    O2 = O.reshape(b, s, hd)

    def body(o_hbm, wo_ref, out_ref, obuf, comm, obuf_sems, send_sems,
             recv_sems, credit_sem):
        my = lax.axis_index("i")
        left = (my - 1) % N_DEV
        right = (my + 1) % N_DEV

        barrier_sem = pltpu.get_barrier_semaphore()
        for nbr in (left, right):
            pl.semaphore_signal(barrier_sem, inc=1, device_id=(nbr,),
                                device_id_type=pl.DeviceIdType.MESH)
        pl.semaphore_wait(barrier_sem, 2)

        def load_chunk(c, slot):
            cp = pltpu.make_async_copy(
                o_hbm.at[:, pl.ds(c * s_chunk, s_chunk), :],
                obuf.at[slot],
                obuf_sems.at[slot],
            )
            cp.start()
            return cp

        def partial(slot, bi):
            return jnp.dot(obuf[slot, bi], wo_ref[...],
                           preferred_element_type=jnp.float32)

        c0 = (my - 1) % N_DEV
        load_chunk(c0, 0).wait()
        for bi in range(b):
            comm[0, bi] = partial(0, bi)

        for t in range(N_DEV - 1):
            send_slot = t % 2
            recv_slot = (t + 1) % 2
            if t >= 1:
                pl.semaphore_wait(credit_sem, 1)
            rdma = pltpu.make_async_remote_copy(
                src_ref=comm.at[send_slot],
                dst_ref=comm.at[recv_slot],
                send_sem=send_sems.at[send_slot],
                recv_sem=recv_sems.at[recv_slot],
                device_id=(right,),
                device_id_type=pl.DeviceIdType.MESH,
            )
            rdma.start()
            c_recv = (my - 2 - t) % N_DEV
            cp = load_chunk(c_recv, recv_slot)
            cp.wait()
            rdma.wait()
            if t < N_DEV - 2:
                pl.semaphore_signal(credit_sem, inc=1, device_id=(left,),
                                    device_id_type=pl.DeviceIdType.MESH)
                for bi in range(b):
                    comm[recv_slot, bi] += partial(recv_slot, bi)
            else:
                for bi in range(b):
                    out_ref[bi] = comm[recv_slot, bi] + partial(recv_slot, bi)

        @functools.partial(pl.run_scoped,
                           exit_sem=pltpu.SemaphoreType.REGULAR)
        def _(exit_sem):
            for nbr in (left, right):
                pl.semaphore_signal(exit_sem, inc=1, device_id=(nbr,),
                                    device_id_type=pl.DeviceIdType.MESH)
            pl.semaphore_wait(exit_sem, 2)

    return pl.pallas_call(
        body,
        out_shape=jax.ShapeDtypeStruct((b, s_chunk, n_out), jnp.float32),
        in_specs=[
            pl.BlockSpec(memory_space=pl.ANY),
            pl.BlockSpec(memory_space=pltpu.VMEM),
        ],
        out_specs=pl.BlockSpec(memory_space=pltpu.VMEM),
        scratch_shapes=[
            pltpu.VMEM((2, b, s_chunk, hd), jnp.float32),
            pltpu.VMEM((2, b, s_chunk, n_out), jnp.float32),
            pltpu.SemaphoreType.DMA((2,)),
            pltpu.SemaphoreType.DMA((2,)),
            pltpu.SemaphoreType.DMA((2,)),
            pltpu.SemaphoreType.REGULAR,
        ],
        compiler_params=pltpu.CompilerParams(
            collective_id=0,
            vmem_limit_bytes=64 * 1024 * 1024,
        ),
    )(O2, Wo)


# device time: 720383 ns/iter; 1.8911x vs baseline; 1.8911x over previous
import functools

import jax
import jax.numpy as jnp
from jax import lax
from jax.experimental import pallas as pl
from jax.experimental.pallas import tpu as pltpu

N_DEV = 8
N_SUB = 2


def kernel(O, Wo):
    b, s, h, d = O.shape
    hd = h * d
    n_out = Wo.shape[1]
    nh = n_out // 2
    nq = nh // N_SUB
    s_chunk = s // N_DEV
    O2 = O.reshape(b, s, hd)

    def body(o_hbm, wo_ref, out_ref, obuf, comm_r, comm_l, obuf_sems,
             send_sems_r, recv_sems_r, send_sems_l, recv_sems_l,
             credit_r, credit_l):
        my = lax.axis_index("i")
        left = (my - 1) % N_DEV
        right = (my + 1) % N_DEV

        barrier_sem = pltpu.get_barrier_semaphore()
        for nbr in (left, right):
            pl.semaphore_signal(barrier_sem, inc=1, device_id=(nbr,),
                                device_id_type=pl.DeviceIdType.MESH)
        pl.semaphore_wait(barrier_sem, 2)

        def load_chunk(c, slot):
            cp = pltpu.make_async_copy(
                o_hbm.at[:, pl.ds(c * s_chunk, s_chunk), :],
                obuf.at[slot],
                obuf_sems.at[slot],
            )
            cp.start()
            return cp

        def partial_r(bi, q):
            return jnp.dot(obuf[0, bi], wo_ref[:, q * nq:(q + 1) * nq],
                           preferred_element_type=jnp.float32)

        def partial_l(bi, q):
            return jnp.dot(obuf[1, bi], wo_ref[:, nh + q * nq:nh + (q + 1) * nq],
                           preferred_element_type=jnp.float32)

        def sub_rdma(comm, send_sems, recv_sems, send_slot, recv_slot, q, tgt):
            return pltpu.make_async_remote_copy(
                src_ref=comm.at[send_slot, :, :, pl.ds(q * nq, nq)],
                dst_ref=comm.at[recv_slot, :, :, pl.ds(q * nq, nq)],
                send_sem=send_sems.at[send_slot, q],
                recv_sem=recv_sems.at[recv_slot, q],
                device_id=(tgt,),
                device_id_type=pl.DeviceIdType.MESH,
            )

        load_chunk((my - 1) % N_DEV, 0).wait()
        rdmas = {}
        for q in range(N_SUB):
            for bi in range(b):
                comm_r[0, bi, :, q * nq:(q + 1) * nq] = partial_r(bi, q)
            rdmas[("r", q)] = sub_rdma(comm_r, send_sems_r, recv_sems_r,
                                       0, 1, q, right)
            rdmas[("r", q)].start()
        cp_l = load_chunk((my + 1) % N_DEV, 1)
        cp_l.wait()
        for q in range(N_SUB):
            for bi in range(b):
                comm_l[0, bi, :, q * nq:(q + 1) * nq] = partial_l(bi, q)
            rdmas[("l", q)] = sub_rdma(comm_l, send_sems_l, recv_sems_l,
                                       0, 1, q, left)
            rdmas[("l", q)].start()

        for t in range(N_DEV - 1):
            send_slot = t % 2
            recv_slot = (t + 1) % 2
            cp_r = load_chunk((my - 2 - t) % N_DEV, 0)
            cp_l = load_chunk((my + 2 + t) % N_DEV, 1)
            cp_r.wait()
            cp_l.wait()

            last = t == N_DEV - 2

            for q in range(N_SUB):
                rdmas[("r", q)].wait()
                if not last:
                    for bi in range(b):
                        comm_r[recv_slot, bi, :, q * nq:(q + 1) * nq] += (
                            partial_r(bi, q))
                else:
                    for bi in range(b):
                        out_ref[bi, :, q * nq:(q + 1) * nq] = (
                            comm_r[recv_slot, bi, :, q * nq:(q + 1) * nq]
                            + partial_r(bi, q))
                rdmas[("l", q)].wait()
                if not last:
                    for bi in range(b):
                        comm_l[recv_slot, bi, :, q * nq:(q + 1) * nq] += (
                            partial_l(bi, q))
                else:
                    for bi in range(b):
                        out_ref[bi, :, nh + q * nq:nh + (q + 1) * nq] = (
                            comm_l[recv_slot, bi, :, q * nq:(q + 1) * nq]
                            + partial_l(bi, q))

            if not last:
                pl.semaphore_signal(credit_r, inc=1, device_id=(left,),
                                    device_id_type=pl.DeviceIdType.MESH)
                pl.semaphore_signal(credit_l, inc=1, device_id=(right,),
                                    device_id_type=pl.DeviceIdType.MESH)
                pl.semaphore_wait(credit_r, 1)
                for q in range(N_SUB):
                    rdmas[("r", q)] = sub_rdma(
                        comm_r, send_sems_r, recv_sems_r,
                        recv_slot, send_slot, q, right)
                    rdmas[("r", q)].start()
                pl.semaphore_wait(credit_l, 1)
                for q in range(N_SUB):
                    rdmas[("l", q)] = sub_rdma(
                        comm_l, send_sems_l, recv_sems_l,
                        recv_slot, send_slot, q, left)
                    rdmas[("l", q)].start()

        @functools.partial(pl.run_scoped,
                           exit_sem=pltpu.SemaphoreType.REGULAR)
        def _(exit_sem):
            for nbr in (left, right):
                pl.semaphore_signal(exit_sem, inc=1, device_id=(nbr,),
                                    device_id_type=pl.DeviceIdType.MESH)
            pl.semaphore_wait(exit_sem, 2)

    return pl.pallas_call(
        body,
        out_shape=jax.ShapeDtypeStruct((b, s_chunk, n_out), jnp.float32),
        in_specs=[
            pl.BlockSpec(memory_space=pl.ANY),
            pl.BlockSpec(memory_space=pltpu.VMEM),
        ],
        out_specs=pl.BlockSpec(memory_space=pltpu.VMEM),
        scratch_shapes=[
            pltpu.VMEM((2, b, s_chunk, hd), jnp.float32),
            pltpu.VMEM((2, b, s_chunk, nh), jnp.float32),
            pltpu.VMEM((2, b, s_chunk, nh), jnp.float32),
            pltpu.SemaphoreType.DMA((2,)),
            pltpu.SemaphoreType.DMA((2, N_SUB)),
            pltpu.SemaphoreType.DMA((2, N_SUB)),
            pltpu.SemaphoreType.DMA((2, N_SUB)),
            pltpu.SemaphoreType.DMA((2, N_SUB)),
            pltpu.SemaphoreType.REGULAR,
            pltpu.SemaphoreType.REGULAR,
        ],
        compiler_params=pltpu.CompilerParams(
            collective_id=0,
            vmem_limit_bytes=64 * 1024 * 1024,
        ),
    )(O2, Wo)


# device time: 682511 ns/iter; 1.9960x vs baseline; 1.0555x over previous
import functools

import jax
import jax.numpy as jnp
from jax import lax
from jax.experimental import pallas as pl
from jax.experimental.pallas import tpu as pltpu

N_DEV = 8
N_SUB = 2


def kernel(O, Wo):
    b, s, h, d = O.shape
    hd = h * d
    n_out = Wo.shape[1]
    nh = n_out // 2
    nq = nh // N_SUB
    s_chunk = s // N_DEV
    O2 = O.reshape(b, s, hd)

    def body(o_hbm, wo_ref, out_ref, obuf, comm_r, comm_l, obuf_sems,
             send_sems_r, recv_sems_r, send_sems_l, recv_sems_l,
             credit_r, credit_l):
        my = lax.axis_index("i")
        left = (my - 1) % N_DEV
        right = (my + 1) % N_DEV

        barrier_sem = pltpu.get_barrier_semaphore()
        for nbr in (left, right):
            pl.semaphore_signal(barrier_sem, inc=1, device_id=(nbr,),
                                device_id_type=pl.DeviceIdType.MESH)
        pl.semaphore_wait(barrier_sem, 2)

        def load_chunk(c, slot):
            cp = pltpu.make_async_copy(
                o_hbm.at[:, pl.ds(c * s_chunk, s_chunk), :],
                obuf.at[slot],
                obuf_sems.at[slot],
            )
            cp.start()
            return cp

        def partial_r(bi, q):
            return jnp.dot(obuf[0, bi], wo_ref[:, q * nq:(q + 1) * nq],
                           preferred_element_type=jnp.float32)

        def partial_l(bi, q):
            return jnp.dot(obuf[1, bi], wo_ref[:, nh + q * nq:nh + (q + 1) * nq],
                           preferred_element_type=jnp.float32)

        def sub_rdma(comm, send_sems, recv_sems, send_slot, recv_slot, q, tgt):
            return pltpu.make_async_remote_copy(
                src_ref=comm.at[send_slot, :, :, pl.ds(q * nq, nq)],
                dst_ref=comm.at[recv_slot, :, :, pl.ds(q * nq, nq)],
                send_sem=send_sems.at[send_slot, q],
                recv_sem=recv_sems.at[recv_slot, q],
                device_id=(tgt,),
                device_id_type=pl.DeviceIdType.MESH,
            )

        cp_r = load_chunk((my - 1) % N_DEV, 0)
        cp_l = load_chunk((my + 1) % N_DEV, 1)
        cp_r.wait()
        cp_l.wait()
        rdmas = {}
        for q in range(N_SUB):
            for bi in range(b):
                comm_r[0, bi, :, q * nq:(q + 1) * nq] = partial_r(bi, q)
            rdmas[("r", q)] = sub_rdma(comm_r, send_sems_r, recv_sems_r,
                                       0, 1, q, right)
            rdmas[("r", q)].start()
            for bi in range(b):
                comm_l[0, bi, :, q * nq:(q + 1) * nq] = partial_l(bi, q)
            rdmas[("l", q)] = sub_rdma(comm_l, send_sems_l, recv_sems_l,
                                       0, 1, q, left)
            rdmas[("l", q)].start()

        for t in range(N_DEV - 1):
            send_slot = t % 2
            recv_slot = (t + 1) % 2
            cp_r = load_chunk((my - 2 - t) % N_DEV, 0)
            cp_l = load_chunk((my + 2 + t) % N_DEV, 1)
            cp_r.wait()
            cp_l.wait()

            last = t == N_DEV - 2

            for q in range(N_SUB):
                rdmas[("r", q)].wait()
                if not last:
                    pl.semaphore_signal(credit_r, inc=1, device_id=(left,),
                                        device_id_type=pl.DeviceIdType.MESH)
                    for bi in range(b):
                        comm_r[recv_slot, bi, :, q * nq:(q + 1) * nq] += (
                            partial_r(bi, q))
                    pl.semaphore_wait(credit_r, 1)
                    rdmas[("r", q)] = sub_rdma(
                        comm_r, send_sems_r, recv_sems_r,
                        recv_slot, send_slot, q, right)
                    rdmas[("r", q)].start()
                else:
                    for bi in range(b):
                        out_ref[bi, :, q * nq:(q + 1) * nq] = (
                            comm_r[recv_slot, bi, :, q * nq:(q + 1) * nq]
                            + partial_r(bi, q))
                rdmas[("l", q)].wait()
                if not last:
                    pl.semaphore_signal(credit_l, inc=1, device_id=(right,),
                                        device_id_type=pl.DeviceIdType.MESH)
                    for bi in range(b):
                        comm_l[recv_slot, bi, :, q * nq:(q + 1) * nq] += (
                            partial_l(bi, q))
                    pl.semaphore_wait(credit_l, 1)
                    rdmas[("l", q)] = sub_rdma(
                        comm_l, send_sems_l, recv_sems_l,
                        recv_slot, send_slot, q, left)
                    rdmas[("l", q)].start()
                else:
                    for bi in range(b):
                        out_ref[bi, :, nh + q * nq:nh + (q + 1) * nq] = (
                            comm_l[recv_slot, bi, :, q * nq:(q + 1) * nq]
                            + partial_l(bi, q))

        @functools.partial(pl.run_scoped,
                           exit_sem=pltpu.SemaphoreType.REGULAR)
        def _(exit_sem):
            for nbr in (left, right):
                pl.semaphore_signal(exit_sem, inc=1, device_id=(nbr,),
                                    device_id_type=pl.DeviceIdType.MESH)
            pl.semaphore_wait(exit_sem, 2)

    return pl.pallas_call(
        body,
        out_shape=jax.ShapeDtypeStruct((b, s_chunk, n_out), jnp.float32),
        in_specs=[
            pl.BlockSpec(memory_space=pl.ANY),
            pl.BlockSpec(memory_space=pltpu.VMEM),
        ],
        out_specs=pl.BlockSpec(memory_space=pltpu.VMEM),
        scratch_shapes=[
            pltpu.VMEM((2, b, s_chunk, hd), jnp.float32),
            pltpu.VMEM((2, b, s_chunk, nh), jnp.float32),
            pltpu.VMEM((2, b, s_chunk, nh), jnp.float32),
            pltpu.SemaphoreType.DMA((2,)),
            pltpu.SemaphoreType.DMA((2, N_SUB)),
            pltpu.SemaphoreType.DMA((2, N_SUB)),
            pltpu.SemaphoreType.DMA((2, N_SUB)),
            pltpu.SemaphoreType.DMA((2, N_SUB)),
            pltpu.SemaphoreType.REGULAR,
            pltpu.SemaphoreType.REGULAR,
        ],
        compiler_params=pltpu.CompilerParams(
            collective_id=0,
            vmem_limit_bytes=64 * 1024 * 1024,
        ),
    )(O2, Wo)


# device time: 680944 ns/iter; 2.0006x vs baseline; 1.0023x over previous
import functools

import jax
import jax.numpy as jnp
from jax import lax
from jax.experimental import pallas as pl
from jax.experimental.pallas import tpu as pltpu

N_DEV = 8
N_SUB = 4


def kernel(O, Wo):
    b, s, h, d = O.shape
    hd = h * d
    n_out = Wo.shape[1]
    nh = n_out // 2
    nq = nh // N_SUB
    s_chunk = s // N_DEV
    O2 = O.reshape(b, s, hd)

    def body(o_hbm, wo_ref, out_ref, obuf, comm_r, comm_l, obuf_sems,
             send_sems_r, recv_sems_r, send_sems_l, recv_sems_l,
             credit_r, credit_l):
        my = lax.axis_index("i")
        left = (my - 1) % N_DEV
        right = (my + 1) % N_DEV

        barrier_sem = pltpu.get_barrier_semaphore()
        for nbr in (left, right):
            pl.semaphore_signal(barrier_sem, inc=1, device_id=(nbr,),
                                device_id_type=pl.DeviceIdType.MESH)
        pl.semaphore_wait(barrier_sem, 2)

        def load_chunk(c, slot):
            cp = pltpu.make_async_copy(
                o_hbm.at[:, pl.ds(c * s_chunk, s_chunk), :],
                obuf.at[slot],
                obuf_sems.at[slot],
            )
            cp.start()
            return cp

        def partial_r(bi, q):
            return jnp.dot(obuf[0, bi], wo_ref[:, q * nq:(q + 1) * nq],
                           preferred_element_type=jnp.float32)

        def partial_l(bi, q):
            return jnp.dot(obuf[1, bi], wo_ref[:, nh + q * nq:nh + (q + 1) * nq],
                           preferred_element_type=jnp.float32)

        def sub_rdma(comm, send_sems, recv_sems, send_slot, recv_slot, q, tgt):
            return pltpu.make_async_remote_copy(
                src_ref=comm.at[send_slot, :, :, pl.ds(q * nq, nq)],
                dst_ref=comm.at[recv_slot, :, :, pl.ds(q * nq, nq)],
                send_sem=send_sems.at[send_slot, q],
                recv_sem=recv_sems.at[recv_slot, q],
                device_id=(tgt,),
                device_id_type=pl.DeviceIdType.MESH,
            )

        cp_r = load_chunk((my - 1) % N_DEV, 0)
        cp_l = load_chunk((my + 1) % N_DEV, 1)
        cp_r.wait()
        cp_l.wait()
        rdmas = {}
        for q in range(N_SUB):
            for bi in range(b):
                comm_r[0, bi, :, q * nq:(q + 1) * nq] = partial_r(bi, q)
            rdmas[("r", q)] = sub_rdma(comm_r, send_sems_r, recv_sems_r,
                                       0, 1, q, right)
            rdmas[("r", q)].start()
            for bi in range(b):
                comm_l[0, bi, :, q * nq:(q + 1) * nq] = partial_l(bi, q)
            rdmas[("l", q)] = sub_rdma(comm_l, send_sems_l, recv_sems_l,
                                       0, 1, q, left)
            rdmas[("l", q)].start()

        for t in range(N_DEV - 1):
            send_slot = t % 2
            recv_slot = (t + 1) % 2
            cp_r = load_chunk((my - 2 - t) % N_DEV, 0)
            cp_l = load_chunk((my + 2 + t) % N_DEV, 1)
            cp_r.wait()
            cp_l.wait()

            last = t == N_DEV - 2

            for q in range(N_SUB):
                rdmas[("r", q)].wait()
                if not last:
                    pl.semaphore_signal(credit_r, inc=1, device_id=(left,),
                                        device_id_type=pl.DeviceIdType.MESH)
                    for bi in range(b):
                        comm_r[recv_slot, bi, :, q * nq:(q + 1) * nq] += (
                            partial_r(bi, q))
                    pl.semaphore_wait(credit_r, 1)
                    rdmas[("r", q)] = sub_rdma(
                        comm_r, send_sems_r, recv_sems_r,
                        recv_slot, send_slot, q, right)
                    rdmas[("r", q)].start()
                else:
                    for bi in range(b):
                        out_ref[bi, :, q * nq:(q + 1) * nq] = (
                            comm_r[recv_slot, bi, :, q * nq:(q + 1) * nq]
                            + partial_r(bi, q))
                rdmas[("l", q)].wait()
                if not last:
                    pl.semaphore_signal(credit_l, inc=1, device_id=(right,),
                                        device_id_type=pl.DeviceIdType.MESH)
                    for bi in range(b):
                        comm_l[recv_slot, bi, :, q * nq:(q + 1) * nq] += (
                            partial_l(bi, q))
                    pl.semaphore_wait(credit_l, 1)
                    rdmas[("l", q)] = sub_rdma(
                        comm_l, send_sems_l, recv_sems_l,
                        recv_slot, send_slot, q, left)
                    rdmas[("l", q)].start()
                else:
                    for bi in range(b):
                        out_ref[bi, :, nh + q * nq:nh + (q + 1) * nq] = (
                            comm_l[recv_slot, bi, :, q * nq:(q + 1) * nq]
                            + partial_l(bi, q))

        @functools.partial(pl.run_scoped,
                           exit_sem=pltpu.SemaphoreType.REGULAR)
        def _(exit_sem):
            for nbr in (left, right):
                pl.semaphore_signal(exit_sem, inc=1, device_id=(nbr,),
                                    device_id_type=pl.DeviceIdType.MESH)
            pl.semaphore_wait(exit_sem, 2)

    return pl.pallas_call(
        body,
        out_shape=jax.ShapeDtypeStruct((b, s_chunk, n_out), jnp.float32),
        in_specs=[
            pl.BlockSpec(memory_space=pl.ANY),
            pl.BlockSpec(memory_space=pltpu.VMEM),
        ],
        out_specs=pl.BlockSpec(memory_space=pltpu.VMEM),
        scratch_shapes=[
            pltpu.VMEM((2, b, s_chunk, hd), jnp.float32),
            pltpu.VMEM((2, b, s_chunk, nh), jnp.float32),
            pltpu.VMEM((2, b, s_chunk, nh), jnp.float32),
            pltpu.SemaphoreType.DMA((2,)),
            pltpu.SemaphoreType.DMA((2, N_SUB)),
            pltpu.SemaphoreType.DMA((2, N_SUB)),
            pltpu.SemaphoreType.DMA((2, N_SUB)),
            pltpu.SemaphoreType.DMA((2, N_SUB)),
            pltpu.SemaphoreType.REGULAR,
            pltpu.SemaphoreType.REGULAR,
        ],
        compiler_params=pltpu.CompilerParams(
            collective_id=0,
            vmem_limit_bytes=64 * 1024 * 1024,
        ),
    )(O2, Wo)
